# baseline (speedup 1.0000x reference)
import functools

import jax
import jax.numpy as jnp
from jax import lax
from jax.experimental import pallas as pl
from jax.experimental.pallas import tpu as pltpu
from jax.experimental.pallas import tpu_sc as plsc

NC = 2
NS = 16
L = 16
NW = NC * NS

C = 3
N = 32
V = 64 * 64 * 64
VW = V // NW
CH = 2048
NCHUNK = VW // CH
GROUPS = CH // L


def _body(embed_hbm, coefs_hbm, out_hbm, e_v, out_v, coefs_v, in_sem, out_sem):
    wid = lax.axis_index("s") * NC + lax.axis_index("c")
    base = pl.multiple_of(wid * VW, VW)

    pltpu.sync_copy(coefs_hbm, coefs_v)
    is0 = coefs_v[pl.ds((4 * N + 0) * L, L)]
    is1 = coefs_v[pl.ds((4 * N + 1) * L, L)]
    is2 = coefs_v[pl.ds((4 * N + 2) * L, L)]

    for ci in range(NCHUNK):
        cb = base + ci * CH
        cps = [
            pltpu.async_copy(
                embed_hbm.at[pl.ds(c * V + cb, CH)],
                e_v.at[pl.ds(c * CH, CH)],
                in_sem,
            )
            for c in range(C)
        ]
        for cp in cps:
            cp.wait()

        def group(g, _):
            e0 = e_v[pl.ds(g * L, L)]
            e1 = e_v[pl.ds(CH + g * L, L)]
            e2 = e_v[pl.ds(2 * CH + g * L, L)]
            s0 = e0 * is0
            s1 = e1 * is1
            s2 = e2 * is2
            q = e0 * s0 + e1 * s1 + e2 * s2
            for n in range(N):
                b0 = coefs_v[pl.ds((0 * N + n) * L, L)]
                b1 = coefs_v[pl.ds((1 * N + n) * L, L)]
                b2 = coefs_v[pl.ds((2 * N + n) * L, L)]
                rn = coefs_v[pl.ds((3 * N + n) * L, L)]
                acc = (q + rn) + (s0 * b0 + s1 * b1 + s2 * b2)
                out_v[pl.ds(n * CH + g * L, L)] = acc
            return 0

        lax.fori_loop(0, GROUPS, group, 0)

        ops = [
            pltpu.async_copy(
                out_v.at[pl.ds(n * CH, CH)],
                out_hbm.at[pl.ds(n * V + cb, CH)],
                out_sem,
            )
            for n in range(N)
        ]
        for op in ops:
            op.wait()


@jax.jit
def kernel(embed, sigma, centroid):
    inv_s = 1.0 / (sigma + 1e-16)
    b = (-2.0 * centroid).T
    r = jnp.sum(centroid * centroid * inv_s[None, :], axis=1)
    rows = jnp.concatenate([b.reshape(3 * N), r, inv_s])
    coefs = jnp.broadcast_to(rows[:, None], (4 * N + 3, L)).reshape(-1)

    mesh = plsc.VectorSubcoreMesh(
        core_axis_name="c", subcore_axis_name="s", num_cores=NC, num_subcores=NS
    )
    out = pl.kernel(
        _body,
        out_type=jax.ShapeDtypeStruct((N * V,), jnp.float32),
        mesh=mesh,
        scratch_types=[
            pltpu.VMEM((C * CH,), jnp.float32),
            pltpu.VMEM((N * CH,), jnp.float32),
            pltpu.VMEM(((4 * N + 3) * L,), jnp.float32),
            pltpu.SemaphoreType.DMA,
            pltpu.SemaphoreType.DMA,
        ],
    )(embed.reshape(C * V), coefs)
    return out.reshape(N, 64, 64, 64)


# speedup vs baseline: 2.0939x; 2.0939x over previous
import jax
import jax.numpy as jnp
from jax import lax
from jax.experimental import pallas as pl
from jax.experimental.pallas import tpu as pltpu
from jax.experimental.pallas import tpu_sc as plsc

NC = 2
NS = 16
L = 16
NW = NC * NS

C = 3
N = 32
V = 64 * 64 * 64
VW = V // NW
CH = 2048
NCHUNK = VW // CH
GROUPS = CH // L
NB = 8
NBLK = N // NB


def _body(embed_hbm, coefs_hbm, out_hbm, e_v, sq_v, o_v, coefs_v, in_sem, out_sem):
    wid = lax.axis_index("s") * NC + lax.axis_index("c")
    base = pl.multiple_of(wid * VW, VW)

    pltpu.sync_copy(coefs_hbm, coefs_v)
    is_ = [coefs_v[pl.ds((4 * N + c) * L, L)] for c in range(C)]

    def fire_in(ci, p):
        cb = base + ci * CH
        return [
            pltpu.async_copy(
                embed_hbm.at[c, pl.ds(cb, CH)],
                e_v.at[pl.ds((p * C + c) * CH, CH)],
                in_sem,
            )
            for c in range(C)
        ]

    pending_in = fire_in(0, 0)
    out_pending = [None, None]

    for ci in range(NCHUNK):
        p = ci % 2
        cb = base + ci * CH
        for h in pending_in:
            h.wait()
        if ci + 1 < NCHUNK:
            pending_in = fire_in(ci + 1, 1 - p)

        @plsc.parallel_loop(0, GROUPS, unroll=2)
        def stage1(g):
            e0 = e_v[pl.ds((p * C + 0) * CH + g * L, L)]
            e1 = e_v[pl.ds((p * C + 1) * CH + g * L, L)]
            e2 = e_v[pl.ds((p * C + 2) * CH + g * L, L)]
            s0 = e0 * is_[0]
            s1 = e1 * is_[1]
            s2 = e2 * is_[2]
            sq_v[pl.ds(0 * CH + g * L, L)] = s0
            sq_v[pl.ds(1 * CH + g * L, L)] = s1
            sq_v[pl.ds(2 * CH + g * L, L)] = s2
            sq_v[pl.ds(3 * CH + g * L, L)] = e0 * s0 + e1 * s1 + e2 * s2

        for nb in range(NBLK):
            ob = nb % 2
            if out_pending[ob] is not None:
                for h in out_pending[ob]:
                    h.wait()
            ns = [nb * NB + j for j in range(NB)]
            b0 = [coefs_v[pl.ds((0 * N + n) * L, L)] for n in ns]
            b1 = [coefs_v[pl.ds((1 * N + n) * L, L)] for n in ns]
            b2 = [coefs_v[pl.ds((2 * N + n) * L, L)] for n in ns]
            rn = [coefs_v[pl.ds((3 * N + n) * L, L)] for n in ns]

            @plsc.parallel_loop(0, GROUPS, unroll=2)
            def stage2(g):
                s0 = sq_v[pl.ds(0 * CH + g * L, L)]
                s1 = sq_v[pl.ds(1 * CH + g * L, L)]
                s2 = sq_v[pl.ds(2 * CH + g * L, L)]
                q = sq_v[pl.ds(3 * CH + g * L, L)]
                for j in range(NB):
                    a = s0 * b0[j] + rn[j]
                    a = s1 * b1[j] + a
                    a = s2 * b2[j] + a
                    o_v[pl.ds((ob * NB + j) * CH + g * L, L)] = q + a

            out_pending[ob] = [
                pltpu.async_copy(
                    o_v.at[pl.ds((ob * NB + j) * CH, CH)],
                    out_hbm.at[ns[j], pl.ds(cb, CH)],
                    out_sem,
                )
                for j in range(NB)
            ]

    for ob in range(2):
        if out_pending[ob] is not None:
            for h in out_pending[ob]:
                h.wait()


@jax.jit
def kernel(embed, sigma, centroid):
    inv_s = 1.0 / (sigma + 1e-16)
    b = (-2.0 * centroid).T
    r = jnp.sum(centroid * centroid * inv_s[None, :], axis=1)
    rows = jnp.concatenate([b.reshape(3 * N), r, inv_s])
    coefs = jnp.broadcast_to(rows[:, None], (4 * N + 3, L)).reshape(-1)

    mesh = plsc.VectorSubcoreMesh(
        core_axis_name="c", subcore_axis_name="s", num_cores=NC, num_subcores=NS
    )
    out = pl.kernel(
        _body,
        out_type=jax.ShapeDtypeStruct((N, V), jnp.float32),
        mesh=mesh,
        compiler_params=pltpu.CompilerParams(use_tc_tiling_on_sc=False),
        scratch_types=[
            pltpu.VMEM((2 * C * CH,), jnp.float32),
            pltpu.VMEM((4 * CH,), jnp.float32),
            pltpu.VMEM((2 * NB * CH,), jnp.float32),
            pltpu.VMEM(((4 * N + 3) * L,), jnp.float32),
            pltpu.SemaphoreType.DMA,
            pltpu.SemaphoreType.DMA,
        ],
    )(embed.reshape(C, V), coefs)
    return out.reshape(N, 64, 64, 64)


# speedup vs baseline: 4.6160x; 2.2045x over previous
import jax
import jax.numpy as jnp
from jax import lax
from jax.experimental import pallas as pl
from jax.experimental.pallas import tpu as pltpu
from jax.experimental.pallas import tpu_sc as plsc

NC = 2
NS = 16
L = 16
NW = NC * NS

C = 3
N = 32
X = 64
YZ = 64 * 64
XPW = X // NW
GROUPS = YZ // L
NB = 4
NBLK = N // NB


def _body(embed_hbm, coefs_hbm, out_hbm, e_v, sq_v, o_v, coefs_v, in_sem, out_sem):
    wid = lax.axis_index("s") * NC + lax.axis_index("c")

    pltpu.sync_copy(coefs_hbm, coefs_v)
    is_ = [coefs_v[pl.ds((4 * N + c) * L, L)] for c in range(C)]

    def fire_in(x):
        return [
            pltpu.async_copy(embed_hbm.at[c, x], e_v.at[c], in_sem)
            for c in range(C)
        ]

    pending_in = fire_in(XPW * wid)
    out_pending = [None, None]

    for xi in range(XPW):
        x = XPW * wid + xi
        for h in pending_in:
            h.wait()

        @plsc.parallel_loop(0, GROUPS, unroll=2)
        def stage1(g):
            y = g // 4
            z = (g % 4) * L
            e0 = e_v[0, y, pl.ds(z, L)]
            e1 = e_v[1, y, pl.ds(z, L)]
            e2 = e_v[2, y, pl.ds(z, L)]
            s0 = e0 * is_[0]
            s1 = e1 * is_[1]
            s2 = e2 * is_[2]
            sq_v[0, y, pl.ds(z, L)] = s0
            sq_v[1, y, pl.ds(z, L)] = s1
            sq_v[2, y, pl.ds(z, L)] = s2
            sq_v[3, y, pl.ds(z, L)] = e0 * s0 + e1 * s1 + e2 * s2

        if xi + 1 < XPW:
            pending_in = fire_in(x + 1)

        for nb in range(NBLK):
            ob = nb % 2
            if out_pending[ob] is not None:
                for h in out_pending[ob]:
                    h.wait()
            ns = [nb * NB + j for j in range(NB)]
            b0 = [coefs_v[pl.ds((0 * N + n) * L, L)] for n in ns]
            b1 = [coefs_v[pl.ds((1 * N + n) * L, L)] for n in ns]
            b2 = [coefs_v[pl.ds((2 * N + n) * L, L)] for n in ns]
            rn = [coefs_v[pl.ds((3 * N + n) * L, L)] for n in ns]

            @plsc.parallel_loop(0, GROUPS, unroll=2)
            def stage2(g):
                y = g // 4
                z = (g % 4) * L
                s0 = sq_v[0, y, pl.ds(z, L)]
                s1 = sq_v[1, y, pl.ds(z, L)]
                s2 = sq_v[2, y, pl.ds(z, L)]
                q = sq_v[3, y, pl.ds(z, L)]
                for j in range(NB):
                    a = s0 * b0[j] + rn[j]
                    a = s1 * b1[j] + a
                    a = s2 * b2[j] + a
                    o_v[ob, j, y, pl.ds(z, L)] = q + a

            out_pending[ob] = [
                pltpu.async_copy(
                    o_v.at[ob, j], out_hbm.at[ns[j], x], out_sem
                )
                for j in range(NB)
            ]

    for ob in range(2):
        if out_pending[ob] is not None:
            for h in out_pending[ob]:
                h.wait()


@jax.jit
def kernel(embed, sigma, centroid):
    inv_s = 1.0 / (sigma + 1e-16)
    b = (-2.0 * centroid).T
    r = jnp.sum(centroid * centroid * inv_s[None, :], axis=1)
    rows = jnp.concatenate([b.reshape(3 * N), r, inv_s])
    coefs = jnp.broadcast_to(rows[:, None], (4 * N + 3, L)).reshape(-1)

    mesh = plsc.VectorSubcoreMesh(
        core_axis_name="c", subcore_axis_name="s", num_cores=NC, num_subcores=NS
    )
    return pl.kernel(
        _body,
        out_type=jax.ShapeDtypeStruct((N, X, 64, 64), jnp.float32),
        mesh=mesh,
        compiler_params=pltpu.CompilerParams(use_tc_tiling_on_sc=True),
        scratch_types=[
            pltpu.VMEM((C, 64, 64), jnp.float32),
            pltpu.VMEM((4, 64, 64), jnp.float32),
            pltpu.VMEM((2, NB, 64, 64), jnp.float32),
            pltpu.VMEM(((4 * N + 3) * L,), jnp.float32),
            pltpu.SemaphoreType.DMA,
            pltpu.SemaphoreType.DMA,
        ],
    )(embed, coefs)
